# baseline (speedup 1.0000x reference)
import functools

import jax
import jax.numpy as jnp
from jax import lax
from jax.experimental import pallas as pl
from jax.experimental.pallas import tpu as pltpu
from jax.experimental.pallas import tpu_sc as plsc

BATCH = 16384
DIM = 32
L = 16
NC, NS = 2, 16
NW = NC * NS
BPW = BATCH // NW
CHUNK = 128
NCHUNK = BPW // CHUNK


def _dyn_gather(x, idx):
    return lax.gather(
        x, idx[:, None],
        dimension_numbers=lax.GatherDimensionNumbers(
            offset_dims=(), collapsed_slice_dims=(0,), start_index_map=(0,)),
        slice_sizes=(1,),
        mode=lax.GatherScatterMode.PROMISE_IN_BOUNDS)


def _sc_body(inp_hbm, ut_hbm, mt_hbm, out_hbm,
             inp_v, uidx_v, midx_v, urows_v, mrows_v, out_v, sem):
    c = lax.axis_index("c")
    s = lax.axis_index("s")
    wid = s * NC + c
    base = wid * BPW

    lanes = lax.broadcasted_iota(jnp.int32, (L,), 0)

    pltpu.sync_copy(inp_hbm.at[pl.ds(base * 2, BPW * 2)], inp_v)

    half = jnp.where(lanes < 8, lanes, lanes - 8)
    ev = half * 2
    od = ev + 1
    lo_half = lanes < 8
    for g in range(BPW // L):
        a = inp_v[pl.ds(g * 2 * L, L)]
        b = inp_v[pl.ds(g * 2 * L + L, L)]
        u = jnp.where(lo_half, _dyn_gather(a, ev), _dyn_gather(b, ev))
        m = jnp.where(lo_half, _dyn_gather(a, od), _dyn_gather(b, od))
        j, o = g // (CHUNK // L), (g % (CHUNK // L)) * L
        uidx_v[j, pl.ds(o, L)] = u
        midx_v[j, pl.ds(o, L)] = m

    copies = []
    for j in range(NCHUNK):
        copies.append(pltpu.async_copy(
            ut_hbm.at[uidx_v.at[j]],
            urows_v.at[pl.ds(j * CHUNK, CHUNK), :], sem))
        copies.append(pltpu.async_copy(
            mt_hbm.at[midx_v.at[j]],
            mrows_v.at[pl.ds(j * CHUNK, CHUNK), :], sem))
    for cp in copies:
        cp.wait()

    rots = [(lanes + (1 << k)) & (L - 1) for k in range(4)]

    def group_body(g, _):
        acc = jnp.zeros((L,), jnp.float32)
        for r in range(L):
            i = g * L + r
            u0 = urows_v[i, pl.ds(0, L)]
            u1 = urows_v[i, pl.ds(L, L)]
            m0 = mrows_v[i, pl.ds(0, L)]
            m1 = mrows_v[i, pl.ds(L, L)]
            p = u0 * m0 + u1 * m1
            for rot in rots:
                p = p + _dyn_gather(p, rot)
            acc = jnp.where(lanes == r, p, acc)
        out_v[pl.ds(g * L, L)] = acc
        return _

    lax.fori_loop(0, BPW // L, group_body, None)

    pltpu.sync_copy(out_v, out_hbm.at[pl.ds(base, BPW)])


def kernel(inputs, user_table, movie_table):
    mesh = plsc.VectorSubcoreMesh(core_axis_name="c", subcore_axis_name="s")
    f = functools.partial(
        pl.kernel,
        mesh=mesh,
        compiler_params=pltpu.CompilerParams(use_tc_tiling_on_sc=False),
        out_type=jax.ShapeDtypeStruct((BATCH,), jnp.float32),
        scratch_types=[
            pltpu.VMEM((BPW * 2,), jnp.int32),
            pltpu.VMEM((NCHUNK, CHUNK), jnp.int32),
            pltpu.VMEM((NCHUNK, CHUNK), jnp.int32),
            pltpu.VMEM((BPW, DIM), jnp.float32),
            pltpu.VMEM((BPW, DIM), jnp.float32),
            pltpu.VMEM((BPW,), jnp.float32),
            pltpu.SemaphoreType.DMA,
        ],
    )(_sc_body)
    return f(inputs.astype(jnp.int32).reshape(BATCH * 2), user_table,
             movie_table)
